# baseline (speedup 1.0000x reference)
import functools

import jax
import jax.numpy as jnp
from jax import lax
from jax.experimental import pallas as pl
from jax.experimental.pallas import tpu as pltpu
from jax.experimental.pallas import tpu_sc as plsc

NUM_CLASSES = 1000
BATCH = 16384

NC = 2
NS = 16
L = 16
NW = NC * NS

ROWS_PER_W = BATCH // NW
BLOCK = 16
NBLK = ROWS_PER_W // BLOCK

_ZCOLS = tuple(range(0, NUM_CLASSES - L, L)) + (NUM_CLASSES - L,)


def _one_hot_body(x_hbm, out_hbm, idx_v, buf0, buf1, sem0, sem1):
    wid = lax.axis_index("s") * NC + lax.axis_index("c")
    rbase = wid * ROWS_PER_W

    pltpu.sync_copy(x_hbm.at[pl.ds(rbase, ROWS_PER_W)], idx_v)

    zeros16 = jnp.zeros((L,), jnp.float32)
    ones16 = jnp.ones((L,), jnp.float32)
    rowi = lax.iota(jnp.int32, L)

    def _zero_row(r, carry):
        for c in _ZCOLS:
            buf0[r, pl.ds(c, L)] = zeros16
            buf1[r, pl.ds(c, L)] = zeros16
        return carry

    lax.fori_loop(0, BLOCK, _zero_row, 0)

    bufs = (buf0, buf1)
    sems = (sem0, sem1)
    pending = [None, None]
    for it in range(NBLK):
        b = it & 1
        buf = bufs[b]
        if pending[b] is not None:
            pending[b].wait()
            pidx = idx_v[pl.ds((it - 2) * BLOCK, L)]
            plsc.store_scatter(buf, [rowi, pidx], zeros16)
        cidx = idx_v[pl.ds(it * BLOCK, L)]
        plsc.store_scatter(buf, [rowi, cidx], ones16)
        pending[b] = pltpu.async_copy(
            buf, out_hbm.at[pl.ds(rbase + it * BLOCK, BLOCK)], sems[b])
    pending[0].wait()
    pending[1].wait()


_one_hot_sc = functools.partial(
    pl.kernel,
    out_type=jax.ShapeDtypeStruct((BATCH, NUM_CLASSES), jnp.float32),
    mesh=plsc.VectorSubcoreMesh(core_axis_name="c", subcore_axis_name="s"),
    compiler_params=pltpu.CompilerParams(needs_layout_passes=False),
    scratch_types=[
        pltpu.VMEM((ROWS_PER_W,), jnp.int32),
        pltpu.VMEM((BLOCK, NUM_CLASSES), jnp.float32),
        pltpu.VMEM((BLOCK, NUM_CLASSES), jnp.float32),
        pltpu.SemaphoreType.DMA,
        pltpu.SemaphoreType.DMA,
    ],
)(_one_hot_body)


def kernel(x):
    return _one_hot_sc(jnp.reshape(x, (BATCH,)))


# speedup vs baseline: 1.0264x; 1.0264x over previous
import functools

import jax
import jax.numpy as jnp
from jax import lax
from jax.experimental import pallas as pl
from jax.experimental.pallas import tpu as pltpu
from jax.experimental.pallas import tpu_sc as plsc

NUM_CLASSES = 1000
BATCH = 16384

NC = 2
NS = 16
L = 16
NW = NC * NS

ROWS_PER_W = BATCH // NW
BLOCK = 32
NBLK = ROWS_PER_W // BLOCK

_ZCOLS = tuple(range(0, NUM_CLASSES - L, L)) + (NUM_CLASSES - L,)


def _one_hot_body(x_hbm, out_hbm, idx_v, buf0, buf1, sem0, sem1):
    wid = lax.axis_index("s") * NC + lax.axis_index("c")
    rbase = wid * ROWS_PER_W

    pltpu.sync_copy(x_hbm.at[pl.ds(rbase, ROWS_PER_W)], idx_v)

    zeros16 = jnp.zeros((L,), jnp.float32)
    ones16 = jnp.ones((L,), jnp.float32)
    rowi = lax.iota(jnp.int32, L)

    def _zero_row(r, carry):
        for c in _ZCOLS:
            buf0[r, pl.ds(c, L)] = zeros16
            buf1[r, pl.ds(c, L)] = zeros16
        return carry

    lax.fori_loop(0, BLOCK, _zero_row, 0)

    bufs = (buf0, buf1)
    sems = (sem0, sem1)
    pending = [None, None]
    for it in range(NBLK):
        b = it & 1
        buf = bufs[b]
        if pending[b] is not None:
            pending[b].wait()
            for j in range(BLOCK // L):
                pidx = idx_v[pl.ds((it - 2) * BLOCK + j * L, L)]
                plsc.store_scatter(buf, [rowi + j * L, pidx], zeros16)
        for j in range(BLOCK // L):
            cidx = idx_v[pl.ds(it * BLOCK + j * L, L)]
            plsc.store_scatter(buf, [rowi + j * L, cidx], ones16)
        pending[b] = pltpu.async_copy(
            buf, out_hbm.at[pl.ds(rbase + it * BLOCK, BLOCK)], sems[b])
    pending[0].wait()
    pending[1].wait()


_one_hot_sc = functools.partial(
    pl.kernel,
    out_type=jax.ShapeDtypeStruct((BATCH, NUM_CLASSES), jnp.float32),
    mesh=plsc.VectorSubcoreMesh(core_axis_name="c", subcore_axis_name="s"),
    compiler_params=pltpu.CompilerParams(needs_layout_passes=False),
    scratch_types=[
        pltpu.VMEM((ROWS_PER_W,), jnp.int32),
        pltpu.VMEM((BLOCK, NUM_CLASSES), jnp.float32),
        pltpu.VMEM((BLOCK, NUM_CLASSES), jnp.float32),
        pltpu.SemaphoreType.DMA,
        pltpu.SemaphoreType.DMA,
    ],
)(_one_hot_body)


def kernel(x):
    return _one_hot_sc(jnp.reshape(x, (BATCH,)))
